# baseline (speedup 1.0000x reference)
import functools

import jax
import jax.numpy as jnp
from jax import lax
from jax.experimental import pallas as pl
from jax.experimental.pallas import tpu as pltpu
from jax.experimental.pallas import tpu_sc as plsc

_A = 0.4
_B = 0.3
_N = 16777216
_NW = 32
_PW = _N // _NW
_CHUNK = 16384
_NSLABS = _PW // _CHUNK
_NB = 65536
_ROWS = _NB // 128
_LANE = 16

_C = (3.0 - 3.0 * _B) / (_A * _A - _A + 1.0)
_C3 = 3.0 * _C
_C1 = -2.0 * (_A + 1.0) * _C
_C0 = _A * _C + 1.0

_mesh = plsc.VectorSubcoreMesh(core_axis_name="c", subcore_axis_name="s")


@functools.partial(
    pl.kernel,
    out_type=jax.ShapeDtypeStruct((2, _ROWS, 128), jnp.int32),
    mesh=_mesh,
    scratch_types=[
        pltpu.VMEM((_CHUNK,), jnp.float32),
        pltpu.VMEM((_CHUNK,), jnp.float32),
        pltpu.VMEM((_ROWS, 128), jnp.int32),
        pltpu.VMEM((4, 128), jnp.int32),
        pltpu.VMEM_SHARED((_ROWS, 128), jnp.int32),
        pltpu.SemaphoreType.DMA,
        pltpu.SemaphoreType.DMA,
    ],
    compiler_params=pltpu.CompilerParams(needs_layout_passes=False),
)
def _hist_kernel(loss_hbm, cnt_hbm, buf0, buf1, cnt, idxb, shared,
                 sem0, sem1):
    sid = lax.axis_index("s")
    cid = lax.axis_index("c")
    wid = sid * 2 + cid
    base = wid * _PW

    pltpu.async_copy(loss_hbm.at[pl.ds(base, _CHUNK)], buf0, sem0)

    zi = jnp.zeros((_LANE,), jnp.int32)

    @plsc.parallel_loop(0, _NB // _LANE, unroll=4)
    def _(i):
        cnt[i >> 3, pl.ds((i & 7) * _LANE, _LANE)] = zi

    for k in range(4):
        for g in range(8):
            idxb[k, pl.ds(g * _LANE, _LANE)] = (
                lax.iota(jnp.int32, _LANE) + (k * 128 + g * _LANE))

    ones = jnp.ones((_LANE,), jnp.int32)

    def compute(buf):
        @plsc.parallel_loop(0, _CHUNK // _LANE, unroll=8)
        def _(j):
            x = buf[pl.ds(j * _LANE, _LANE)]
            v = lax.bitcast_convert_type(x, jnp.int32)
            r = lax.shift_right_logical(v, 23)
            cc = lax.shift_right_logical(v, 16) & jnp.int32(127)
            plsc.addupdate_scatter(cnt, [r, cc], ones)

    def pair(p, _):
        for b in range(2):
            s = 2 * p + b
            buf, sem = (buf0, sem0) if b == 0 else (buf1, sem1)
            obuf, osem = (buf1, sem1) if b == 0 else (buf0, sem0)

            @pl.when(s + 1 < _NSLABS)
            def _():
                pltpu.async_copy(
                    loss_hbm.at[pl.ds(base + (s + 1) * _CHUNK, _CHUNK)],
                    obuf, osem)

            pltpu.make_async_copy(
                loss_hbm.at[pl.ds(base, _CHUNK)], buf, sem).wait()
            compute(buf)
        return 0

    lax.fori_loop(0, _NSLABS // 2, pair, 0)

    @pl.when(sid == 0)
    def _():
        pltpu.sync_copy(cnt, shared)

    plsc.subcore_barrier()

    @pl.when(sid != 0)
    def _():
        for k in range(4):
            pltpu.sync_copy(cnt.at[pl.ds(k * 128, 128)],
                            shared.at[idxb.at[k]], add=True)

    plsc.subcore_barrier()

    @pl.when(sid == 0)
    def _():
        pltpu.sync_copy(shared, cnt_hbm.at[cid])


def _combine_body(cnt_ref, out_ref):
    cnt = jnp.sum(cnt_ref[...].astype(jnp.float32), axis=0)

    ri = lax.broadcasted_iota(jnp.int32, (_ROWS, 128), 0)
    ci = lax.broadcasted_iota(jnp.int32, (_ROWS, 128), 1)
    tbits = ((ri * 128 + ci) << 16) | jnp.int32(0x8000)
    cb = lax.bitcast_convert_type(tbits, jnp.float32)
    expo = lax.shift_right_logical(tbits, 23) & jnp.int32(0xFF)
    cb = jnp.where(expo == 255, jnp.float32(0.0), cb)
    bsum = cnt * cb

    half = _ROWS // 2
    r1 = lax.broadcasted_iota(jnp.int32, (half, half), 0)
    r2 = lax.broadcasted_iota(jnp.int32, (half, half), 1)
    exch_r = (r1 + r2 == half - 1).astype(jnp.float32)
    c1 = lax.broadcasted_iota(jnp.int32, (128, 128), 0)
    c2 = lax.broadcasted_iota(jnp.int32, (128, 128), 1)
    exch_c = (c1 + c2 == 127).astype(jnp.float32)

    def _flip(x):
        a = lax.dot_general(exch_r, x, (((1,), (0,)), ((), ())),
                            precision=lax.Precision.HIGHEST,
                            preferred_element_type=jnp.float32)
        return lax.dot_general(a, exch_c, (((1,), (0,)), ((), ())),
                               precision=lax.Precision.HIGHEST,
                               preferred_element_type=jnp.float32)

    cnt_s = jnp.concatenate([_flip(cnt[half:]), cnt[:half]], axis=0)

    rows = jnp.sum(cnt_s, axis=1, keepdims=True)
    ri2 = lax.broadcasted_iota(jnp.int32, (_ROWS, _ROWS), 0)
    rj2 = lax.broadcasted_iota(jnp.int32, (_ROWS, _ROWS), 1)
    lower = (rj2 < ri2).astype(jnp.float32)
    row_off = lax.dot_general(
        lower, rows, (((1,), (0,)), ((), ())),
        precision=lax.Precision.HIGHEST,
        preferred_element_type=jnp.float32)
    ci2 = lax.broadcasted_iota(jnp.int32, (128, 128), 0)
    cj2 = lax.broadcasted_iota(jnp.int32, (128, 128), 1)
    upper = (ci2 < cj2).astype(jnp.float32)
    in_row = lax.dot_general(
        cnt_s, upper, (((1,), (0,)), ((), ())),
        precision=lax.Precision.HIGHEST,
        preferred_element_type=jnp.float32)
    rank0 = row_off + in_row

    f = (rank0 + 0.5 * (cnt_s - 1.0)) * (1.0 / _N)
    w_s = (_C3 * f + _C1) * f + _C0

    w_raw = jnp.concatenate([w_s[half:], _flip(w_s[:half])], axis=0)
    out_ref[...] = jnp.sum(w_raw * bsum, keepdims=True) * (1.0 / _N)


_combine = pl.pallas_call(
    _combine_body,
    out_shape=jax.ShapeDtypeStruct((1, 1), jnp.float32),
)


def kernel(loss):
    cnts = _hist_kernel(loss)
    return _combine(cnts)[0, 0]


# speedup vs baseline: 1.1304x; 1.1304x over previous
import functools

import jax
import jax.numpy as jnp
from jax import lax
from jax.experimental import pallas as pl
from jax.experimental.pallas import tpu as pltpu
from jax.experimental.pallas import tpu_sc as plsc

_A = 0.4
_B = 0.3
_N = 16777216
_NW = 32
_PW = _N // _NW
_CHUNK = 32768
_NSLABS = _PW // _CHUNK
_NB = 32768
_ROWS = _NB // 128
_LANE = 16

_C = (3.0 - 3.0 * _B) / (_A * _A - _A + 1.0)
_C3 = 3.0 * _C
_C1 = -2.0 * (_A + 1.0) * _C
_C0 = _A * _C + 1.0

_mesh = plsc.VectorSubcoreMesh(core_axis_name="c", subcore_axis_name="s")


@functools.partial(
    pl.kernel,
    out_type=jax.ShapeDtypeStruct((_NW, _NB), jnp.int32),
    mesh=_mesh,
    scratch_types=[
        pltpu.VMEM((_CHUNK,), jnp.float32),
        pltpu.VMEM((_CHUNK,), jnp.float32),
        pltpu.VMEM((_NB,), jnp.int32),
        pltpu.SemaphoreType.DMA,
        pltpu.SemaphoreType.DMA,
    ],
    compiler_params=pltpu.CompilerParams(needs_layout_passes=False),
)
def _hist_kernel(loss_hbm, cnt_hbm, buf0, buf1, cnt, sem0, sem1):
    wid = lax.axis_index("s") * 2 + lax.axis_index("c")
    base = wid * _PW

    pltpu.async_copy(loss_hbm.at[pl.ds(base, _CHUNK)], buf0, sem0)

    zi = jnp.zeros((_LANE,), jnp.int32)

    @plsc.parallel_loop(0, _NB // _LANE, unroll=4)
    def _(i):
        cnt[pl.ds(i * _LANE, _LANE)] = zi

    ones = jnp.ones((_LANE,), jnp.int32)

    def compute(buf):
        @plsc.parallel_loop(0, _CHUNK // _LANE, unroll=8)
        def _(j):
            x = buf[pl.ds(j * _LANE, _LANE)]
            v = lax.bitcast_convert_type(x, jnp.int32)
            t = lax.shift_right_logical(v, 17)
            plsc.addupdate_scatter(cnt, [t], ones)

    def pair(p, _):
        for b in range(2):
            s = 2 * p + b
            buf, sem = (buf0, sem0) if b == 0 else (buf1, sem1)
            obuf, osem = (buf1, sem1) if b == 0 else (buf0, sem0)

            @pl.when(s + 1 < _NSLABS)
            def _():
                pltpu.async_copy(
                    loss_hbm.at[pl.ds(base + (s + 1) * _CHUNK, _CHUNK)],
                    obuf, osem)

            pltpu.make_async_copy(
                loss_hbm.at[pl.ds(base, _CHUNK)], buf, sem).wait()
            compute(buf)
        return 0

    lax.fori_loop(0, _NSLABS // 2, pair, 0)
    pltpu.sync_copy(cnt, cnt_hbm.at[wid])


def _combine_body(cnt_ref, out_ref):
    cnt = jnp.sum(cnt_ref[...].astype(jnp.float32), axis=0)
    cnt = cnt.reshape(_ROWS, 128)

    ri = lax.broadcasted_iota(jnp.int32, (_ROWS, 128), 0)
    ci = lax.broadcasted_iota(jnp.int32, (_ROWS, 128), 1)
    tbits = ((ri * 128 + ci) << 17) | jnp.int32(0x10000)
    cb = lax.bitcast_convert_type(tbits, jnp.float32)
    expo = lax.shift_right_logical(tbits, 23) & jnp.int32(0xFF)
    cb = jnp.where(expo == 255, jnp.float32(0.0), cb)
    bsum = cnt * cb

    half = _ROWS // 2
    r1 = lax.broadcasted_iota(jnp.int32, (half, half), 0)
    r2 = lax.broadcasted_iota(jnp.int32, (half, half), 1)
    exch_r = (r1 + r2 == half - 1).astype(jnp.float32)
    c1 = lax.broadcasted_iota(jnp.int32, (128, 128), 0)
    c2 = lax.broadcasted_iota(jnp.int32, (128, 128), 1)
    exch_c = (c1 + c2 == 127).astype(jnp.float32)

    def _flip(x):
        a = lax.dot_general(exch_r, x, (((1,), (0,)), ((), ())),
                            precision=lax.Precision.HIGHEST,
                            preferred_element_type=jnp.float32)
        return lax.dot_general(a, exch_c, (((1,), (0,)), ((), ())),
                               precision=lax.Precision.HIGHEST,
                               preferred_element_type=jnp.float32)

    cnt_s = jnp.concatenate([_flip(cnt[half:]), cnt[:half]], axis=0)

    rows = jnp.sum(cnt_s, axis=1, keepdims=True)
    ri2 = lax.broadcasted_iota(jnp.int32, (_ROWS, _ROWS), 0)
    rj2 = lax.broadcasted_iota(jnp.int32, (_ROWS, _ROWS), 1)
    lower = (rj2 < ri2).astype(jnp.float32)
    row_off = lax.dot_general(
        lower, rows, (((1,), (0,)), ((), ())),
        precision=lax.Precision.HIGHEST,
        preferred_element_type=jnp.float32)
    ci2 = lax.broadcasted_iota(jnp.int32, (128, 128), 0)
    cj2 = lax.broadcasted_iota(jnp.int32, (128, 128), 1)
    upper = (ci2 < cj2).astype(jnp.float32)
    in_row = lax.dot_general(
        cnt_s, upper, (((1,), (0,)), ((), ())),
        precision=lax.Precision.HIGHEST,
        preferred_element_type=jnp.float32)
    rank0 = row_off + in_row

    f = (rank0 + 0.5 * (cnt_s - 1.0)) * (1.0 / _N)
    w_s = (_C3 * f + _C1) * f + _C0

    w_raw = jnp.concatenate([w_s[half:], _flip(w_s[:half])], axis=0)
    out_ref[...] = jnp.sum(w_raw * bsum, keepdims=True) * (1.0 / _N)


_combine = pl.pallas_call(
    _combine_body,
    out_shape=jax.ShapeDtypeStruct((1, 1), jnp.float32),
)


def kernel(loss):
    cnts = _hist_kernel(loss)
    return _combine(cnts)[0, 0]
